# Initial kernel scaffold; baseline (speedup 1.0000x reference)
import jax
import jax.numpy as jnp
from jax.experimental import pallas as pl


def kernel(node_scalar, node_vector, edge, edge_diff, edge_dist, rbf_dist, W1, b1, a1, W2, b2, a2, Wf1, bf1, af1, Wf2, bf2, af2):
    raise NotImplementedError("write your pallas kernel here")


# speedup vs baseline: 9.6818x; 9.6818x over previous
import functools

import jax
import jax.numpy as jnp
from jax import lax
from jax.experimental import pallas as pl
from jax.experimental.pallas import tpu as pltpu
from jax.experimental.pallas import tpu_sc as plsc

NC = 2
NS = 16
NW = NC * NS
LANES = 16


def _prelu(x, a):
    return jnp.where(x >= 0, x, a * x)


_BCAST_DNUMS = lax.GatherDimensionNumbers(
    offset_dims=(), collapsed_slice_dims=(0,), start_index_map=(0,))


def _lane_bcast(vec, jj):
    idx = jnp.full((LANES, 1), jj, jnp.int32)
    return lax.gather(vec, idx, _BCAST_DNUMS, (1,),
                      mode=lax.GatherScatterMode.PROMISE_IN_BOUNDS)


def _node_mlp_body(ns, nv0, nv1, nv2, w1, b1, a1, w2, b2, a2,
                   p0, p1, p2, so2, so3):
    h = _prelu(jnp.dot(ns[...], w1[...], preferred_element_type=jnp.float32)
               + b1[...], a1[0, 0])
    y = _prelu(jnp.dot(h, w2[...], preferred_element_type=jnp.float32)
               + b2[...], a2[0, 0])
    H = ns.shape[1]
    so1 = y[:, :H]
    p0[...] = nv0[...] * so1
    p1[...] = nv1[...] * so1
    p2[...] = nv2[...] * so1
    so2[...] = y[:, H:2 * H]
    so3[...] = y[:, 2 * H:]


def _node_mlp(ns, nv0, nv1, nv2, w1, b1, a1, w2, b2, a2):
    n, h = ns.shape
    bn = 400
    grid = (n // bn,)
    row = lambda i: (i, 0)
    fixed = lambda i: (0, 0)
    out = jax.ShapeDtypeStruct((n, h), jnp.float32)
    return pl.pallas_call(
        _node_mlp_body,
        grid=grid,
        in_specs=[
            pl.BlockSpec((bn, h), row),
            pl.BlockSpec((bn, h), row),
            pl.BlockSpec((bn, h), row),
            pl.BlockSpec((bn, h), row),
            pl.BlockSpec((h, h), fixed),
            pl.BlockSpec((1, h), fixed),
            pl.BlockSpec((1, 1), fixed),
            pl.BlockSpec((h, 3 * h), fixed),
            pl.BlockSpec((1, 3 * h), fixed),
            pl.BlockSpec((1, 1), fixed),
        ],
        out_specs=[pl.BlockSpec((bn, h), row)] * 5,
        out_shape=[out] * 5,
    )(ns, nv0, nv1, nv2, w1, b1, a1, w2, b2, a2)


def _filter_mlp_body(rbf, wf1, bf1, af1, wf2, bf2, af2, fw1, fw2, fw3):
    h = _prelu(jnp.dot(rbf[...], wf1[...], preferred_element_type=jnp.float32)
               + bf1[...], af1[0, 0])
    y = _prelu(jnp.dot(h, wf2[...], preferred_element_type=jnp.float32)
               + bf2[...], af2[0, 0])
    H = wf1.shape[1]
    fw1[...] = y[:, :H]
    fw2[...] = y[:, H:2 * H]
    fw3[...] = y[:, 2 * H:]


def _filter_mlp(rbf, wf1, bf1, af1, wf2, bf2, af2):
    e, rb = rbf.shape
    h = wf1.shape[1]
    be = 2000
    grid = (e // be,)
    row = lambda i: (i, 0)
    fixed = lambda i: (0, 0)
    out = jax.ShapeDtypeStruct((e, h), jnp.float32)
    return pl.pallas_call(
        _filter_mlp_body,
        grid=grid,
        in_specs=[
            pl.BlockSpec((be, rb), row),
            pl.BlockSpec((rb, h), fixed),
            pl.BlockSpec((1, h), fixed),
            pl.BlockSpec((1, 1), fixed),
            pl.BlockSpec((h, 3 * h), fixed),
            pl.BlockSpec((1, 3 * h), fixed),
            pl.BlockSpec((1, 1), fixed),
        ],
        out_specs=[pl.BlockSpec((be, h), row)] * 3,
        out_shape=[out] * 3,
    )(rbf, wf1, bf1, af1, wf2, bf2, af2)


ZR = 24


def _zero_acc(zbuf, acc, sid, n, nr8, rem):
    def zb(i, _):
        for j in range(8):
            zbuf[i, pl.ds(LANES * j, LANES)] = jnp.zeros((LANES,), jnp.float32)
        return 0
    lax.fori_loop(0, ZR, zb, 0)
    for r in range(nr8 // ZR):
        pltpu.sync_copy(zbuf, acc.at[pl.ds(sid * nr8 + r * ZR, ZR)])
    if rem:
        @pl.when(sid == NS - 1)
        def _():
            pltpu.sync_copy(zbuf.at[pl.ds(0, rem)],
                            acc.at[pl.ds(NS * nr8, rem)])


def _writeback(acc, out_hbm, core, sid, n, nr8, rem):
    pltpu.sync_copy(acc.at[pl.ds(sid * nr8, nr8)],
                    out_hbm.at[pl.ds(core * n + sid * nr8, nr8)])
    if rem:
        @pl.when(sid == NS - 1)
        def _():
            pltpu.sync_copy(acc.at[pl.ds(NS * nr8, rem)],
                            out_hbm.at[pl.ds(core * n + NS * nr8, rem)])


def _sc_scalar_pass(so2, fw2, src, dst):
    n, h = so2.shape
    e = src.shape[0]
    ew = e // NW
    ch = 80
    nr8 = (n // NS) // 8 * 8
    rem = n - NS * nr8
    mesh = plsc.VectorSubcoreMesh(core_axis_name="c", subcore_axis_name="s",
                                  num_cores=NC, num_subcores=NS)

    @functools.partial(
        pl.kernel, mesh=mesh,
        out_type=jax.ShapeDtypeStruct((NC * n, h), jnp.float32),
        scratch_types=[
            pltpu.VMEM((ch,), jnp.int32),
            pltpu.VMEM((ch,), jnp.int32),
            pltpu.VMEM((ch, h), jnp.float32),
            pltpu.VMEM((ch, h), jnp.float32),
            pltpu.VMEM((ZR, h), jnp.float32),
            pltpu.VMEM_SHARED((n, h), jnp.float32),
            pltpu.SemaphoreType.DMA,
        ],
    )
    def k(so2_hbm, fw2_hbm, src_hbm, dst_hbm, out_hbm,
          dst_v, src_v, rows_v, fw_v, zbuf, acc, sem):
        core = lax.axis_index("c")
        sid = lax.axis_index("s")
        wid = sid * NC + core
        _zero_acc(zbuf, acc, sid, n, nr8, rem)
        plsc.subcore_barrier()

        def chunk(t, _):
            base = wid * ew + t * ch
            pltpu.sync_copy(dst_hbm.at[pl.ds(base, ch)], dst_v)
            pltpu.sync_copy(src_hbm.at[pl.ds(base, ch)], src_v)
            pltpu.sync_copy(fw2_hbm.at[pl.ds(base, ch)], fw_v)
            pltpu.async_copy(so2_hbm.at[dst_v], rows_v, sem).wait()

            def body(i, _):
                for j in range(h // LANES):
                    sl = pl.ds(LANES * j, LANES)
                    rows_v[i, sl] = rows_v[i, sl] * fw_v[i, sl]
                return 0
            lax.fori_loop(0, ch, body, 0)
            pltpu.sync_copy(rows_v, acc.at[src_v], add=True)
            return 0
        lax.fori_loop(0, ew // ch, chunk, 0)
        plsc.subcore_barrier()
        _writeback(acc, out_hbm, core, sid, n, nr8, rem)

    return k(so2, fw2, src, dst)


def _sc_vec_pass(p, so3, fw1, fw3, diff_c, dist, src, dst):
    n, h = p.shape
    e = src.shape[0]
    ew = e // NW
    ch = 80
    nr8 = (n // NS) // 8 * 8
    rem = n - NS * nr8
    mesh = plsc.VectorSubcoreMesh(core_axis_name="c", subcore_axis_name="s",
                                  num_cores=NC, num_subcores=NS)

    @functools.partial(
        pl.kernel, mesh=mesh,
        out_type=jax.ShapeDtypeStruct((NC * n, h), jnp.float32),
        scratch_types=[
            pltpu.VMEM((ch,), jnp.int32),
            pltpu.VMEM((ch,), jnp.int32),
            pltpu.VMEM((ch, h), jnp.float32),
            pltpu.VMEM((ch, h), jnp.float32),
            pltpu.VMEM((ch, h), jnp.float32),
            pltpu.VMEM((ch, h), jnp.float32),
            pltpu.VMEM((ch,), jnp.float32),
            pltpu.VMEM((ch,), jnp.float32),
            pltpu.VMEM((ZR, h), jnp.float32),
            pltpu.VMEM_SHARED((n, h), jnp.float32),
            pltpu.SemaphoreType.DMA,
        ],
    )
    def k(p_hbm, so3_hbm, fw1_hbm, fw3_hbm, diff_hbm, dist_hbm, src_hbm,
          dst_hbm, out_hbm, dst_v, src_v, p_v, so3_v, fw1_v, fw3_v,
          diff_v, dist_v, zbuf, acc, sem):
        core = lax.axis_index("c")
        sid = lax.axis_index("s")
        wid = sid * NC + core
        _zero_acc(zbuf, acc, sid, n, nr8, rem)
        plsc.subcore_barrier()

        def chunk(t, _):
            base = wid * ew + t * ch
            pltpu.sync_copy(dst_hbm.at[pl.ds(base, ch)], dst_v)
            pltpu.sync_copy(src_hbm.at[pl.ds(base, ch)], src_v)
            pltpu.sync_copy(fw1_hbm.at[pl.ds(base, ch)], fw1_v)
            pltpu.sync_copy(fw3_hbm.at[pl.ds(base, ch)], fw3_v)
            pltpu.sync_copy(diff_hbm.at[pl.ds(base, ch)], diff_v)
            pltpu.sync_copy(dist_hbm.at[pl.ds(base, ch)], dist_v)
            pltpu.async_copy(p_hbm.at[dst_v], p_v, sem).wait()
            pltpu.async_copy(so3_hbm.at[dst_v], so3_v, sem).wait()

            def grp(g, _):
                gsl = pl.ds(LANES * g, LANES)
                udv = diff_v[gsl] / dist_v[gsl]
                for jj in range(LANES):
                    i = LANES * g + jj
                    bc = _lane_bcast(udv, jj)
                    for j in range(h // LANES):
                        sl = pl.ds(LANES * j, LANES)
                        p_v[i, sl] = (p_v[i, sl] * fw1_v[i, sl]
                                      + bc * (so3_v[i, sl] * fw3_v[i, sl]))
                return 0
            lax.fori_loop(0, ch // LANES, grp, 0)
            pltpu.sync_copy(p_v, acc.at[src_v], add=True)
            return 0
        lax.fori_loop(0, ew // ch, chunk, 0)
        plsc.subcore_barrier()
        _writeback(acc, out_hbm, core, sid, n, nr8, rem)

    return k(p, so3, fw1, fw3, diff_c, dist, src, dst)


def _final_body(ns, nv0, nv1, nv2, rs, rv0, rv1, rv2,
                os_, ov0, ov1, ov2):
    os_[...] = ns[...] + rs[0] + rs[1]
    ov0[...] = nv0[...] + rv0[0] + rv0[1]
    ov1[...] = nv1[...] + rv1[0] + rv1[1]
    ov2[...] = nv2[...] + rv2[0] + rv2[1]


def _final_add(ns, nv0, nv1, nv2, rs, rv0, rv1, rv2):
    n, h = ns.shape
    bn = 400
    grid = (n // bn,)
    row = lambda i: (i, 0)
    row3 = lambda i: (0, i, 0)
    out = jax.ShapeDtypeStruct((n, h), jnp.float32)
    return pl.pallas_call(
        _final_body,
        grid=grid,
        in_specs=[pl.BlockSpec((bn, h), row)] * 4
                 + [pl.BlockSpec((NC, bn, h), row3)] * 4,
        out_specs=[pl.BlockSpec((bn, h), row)] * 4,
        out_shape=[out] * 4,
    )(ns, nv0, nv1, nv2, rs, rv0, rv1, rv2)


def kernel(node_scalar, node_vector, edge, edge_diff, edge_dist, rbf_dist,
           W1, b1, a1, W2, b2, a2, Wf1, bf1, af1, Wf2, bf2, af2):
    n, h = node_scalar.shape
    e = edge.shape[0]

    src = edge[:, 0].astype(jnp.int32)
    dst = edge[:, 1].astype(jnp.int32)
    nv0 = node_vector[:, 0, :]
    nv1 = node_vector[:, 1, :]
    nv2 = node_vector[:, 2, :]
    d0 = edge_diff[:, 0]
    d1 = edge_diff[:, 1]
    d2 = edge_diff[:, 2]
    b1r = b1.reshape(1, h)
    b2r = b2.reshape(1, 3 * h)
    bf1r = bf1.reshape(1, h)
    bf2r = bf2.reshape(1, 3 * h)
    a1r = a1.reshape(1, 1)
    a2r = a2.reshape(1, 1)
    af1r = af1.reshape(1, 1)
    af2r = af2.reshape(1, 1)

    p0, p1, p2, so2, so3 = _node_mlp(node_scalar, nv0, nv1, nv2,
                                     W1, b1r, a1r, W2, b2r, a2r)
    fw1, fw2, fw3 = _filter_mlp(rbf_dist, Wf1, bf1r, af1r, Wf2, bf2r, af2r)

    rs = _sc_scalar_pass(so2, fw2, src, dst).reshape(NC, n, h)
    rv0 = _sc_vec_pass(p0, so3, fw1, fw3, d0, edge_dist, src, dst).reshape(NC, n, h)
    rv1 = _sc_vec_pass(p1, so3, fw1, fw3, d1, edge_dist, src, dst).reshape(NC, n, h)
    rv2 = _sc_vec_pass(p2, so3, fw1, fw3, d2, edge_dist, src, dst).reshape(NC, n, h)

    os_, ov0, ov1, ov2 = _final_add(node_scalar, nv0, nv1, nv2,
                                    rs, rv0, rv1, rv2)
    out_vector = jnp.stack([ov0, ov1, ov2], axis=1)
    return (os_, out_vector)


# speedup vs baseline: 11.6247x; 1.2007x over previous
import functools

import jax
import jax.numpy as jnp
from jax import lax
from jax.experimental import pallas as pl
from jax.experimental.pallas import tpu as pltpu
from jax.experimental.pallas import tpu_sc as plsc

NC = 2
NS = 16
NW = NC * NS
LANES = 16


def _prelu(x, a):
    return jnp.where(x >= 0, x, a * x)


_BCAST_DNUMS = lax.GatherDimensionNumbers(
    offset_dims=(), collapsed_slice_dims=(0,), start_index_map=(0,))


def _lane_bcast(vec, jj):
    idx = jnp.full((LANES, 1), jj, jnp.int32)
    return lax.gather(vec, idx, _BCAST_DNUMS, (1,),
                      mode=lax.GatherScatterMode.PROMISE_IN_BOUNDS)


def _node_mlp_body(ns, nv0, nv1, nv2, w1, b1, a1, w2, b2, a2,
                   p0, p1, p2, so2, so3):
    h = _prelu(jnp.dot(ns[...], w1[...], preferred_element_type=jnp.float32)
               + b1[...], a1[0, 0])
    y = _prelu(jnp.dot(h, w2[...], preferred_element_type=jnp.float32)
               + b2[...], a2[0, 0])
    H = ns.shape[1]
    so1 = y[:, :H]
    p0[...] = nv0[...] * so1
    p1[...] = nv1[...] * so1
    p2[...] = nv2[...] * so1
    so2[...] = y[:, H:2 * H]
    so3[...] = y[:, 2 * H:]


def _node_mlp(ns, nv0, nv1, nv2, w1, b1, a1, w2, b2, a2):
    n, h = ns.shape
    bn = 400
    grid = (n // bn,)
    row = lambda i: (i, 0)
    fixed = lambda i: (0, 0)
    out = jax.ShapeDtypeStruct((n, h), jnp.float32)
    return pl.pallas_call(
        _node_mlp_body,
        grid=grid,
        in_specs=[
            pl.BlockSpec((bn, h), row),
            pl.BlockSpec((bn, h), row),
            pl.BlockSpec((bn, h), row),
            pl.BlockSpec((bn, h), row),
            pl.BlockSpec((h, h), fixed),
            pl.BlockSpec((1, h), fixed),
            pl.BlockSpec((1, 1), fixed),
            pl.BlockSpec((h, 3 * h), fixed),
            pl.BlockSpec((1, 3 * h), fixed),
            pl.BlockSpec((1, 1), fixed),
        ],
        out_specs=[pl.BlockSpec((bn, h), row)] * 5,
        out_shape=[out] * 5,
    )(ns, nv0, nv1, nv2, w1, b1, a1, w2, b2, a2)


def _filter_mlp_body(rbf, wf1, bf1, af1, wf2, bf2, af2, fw1, fw2, fw3):
    h = _prelu(jnp.dot(rbf[...], wf1[...], preferred_element_type=jnp.float32)
               + bf1[...], af1[0, 0])
    y = _prelu(jnp.dot(h, wf2[...], preferred_element_type=jnp.float32)
               + bf2[...], af2[0, 0])
    H = wf1.shape[1]
    fw1[...] = y[:, :H]
    fw2[...] = y[:, H:2 * H]
    fw3[...] = y[:, 2 * H:]


def _filter_mlp(rbf, wf1, bf1, af1, wf2, bf2, af2):
    e, rb = rbf.shape
    h = wf1.shape[1]
    be = 2048
    grid = (e // be,)
    row = lambda i: (i, 0)
    fixed = lambda i: (0, 0)
    out = jax.ShapeDtypeStruct((e, h), jnp.float32)
    return pl.pallas_call(
        _filter_mlp_body,
        grid=grid,
        in_specs=[
            pl.BlockSpec((be, rb), row),
            pl.BlockSpec((rb, h), fixed),
            pl.BlockSpec((1, h), fixed),
            pl.BlockSpec((1, 1), fixed),
            pl.BlockSpec((h, 3 * h), fixed),
            pl.BlockSpec((1, 3 * h), fixed),
            pl.BlockSpec((1, 1), fixed),
        ],
        out_specs=[pl.BlockSpec((be, h), row)] * 3,
        out_shape=[out] * 3,
    )(rbf, wf1, bf1, af1, wf2, bf2, af2)


def _sc_scalar_pass(so2, fw2, src, dst, zeros):
    n, h = so2.shape
    na = zeros.shape[0]
    e = src.shape[0]
    ew = e // NW
    ch = 32
    nt = ew // ch
    nr = na // NS
    mesh = plsc.VectorSubcoreMesh(core_axis_name="c", subcore_axis_name="s",
                                  num_cores=NC, num_subcores=NS)

    @functools.partial(
        pl.kernel, mesh=mesh,
        out_type=jax.ShapeDtypeStruct((NC * na, h), jnp.float32),
        scratch_types=[
            pltpu.VMEM((ch,), jnp.int32), pltpu.VMEM((ch,), jnp.int32),
            pltpu.VMEM((ch,), jnp.int32), pltpu.VMEM((ch,), jnp.int32),
            pltpu.VMEM((ch, h), jnp.float32), pltpu.VMEM((ch, h), jnp.float32),
            pltpu.VMEM((ch, h), jnp.float32), pltpu.VMEM((ch, h), jnp.float32),
            pltpu.VMEM_SHARED((na, h), jnp.float32),
            pltpu.SemaphoreType.DMA, pltpu.SemaphoreType.DMA,
            pltpu.SemaphoreType.DMA, pltpu.SemaphoreType.DMA,
        ],
    )
    def k(so2_hbm, fw2_hbm, src_hbm, dst_hbm, z_hbm, out_hbm,
          dst0, dst1, src0, src1, r0, r1, f0, f1, acc, sl0, sl1, sg0, sg1):
        core = lax.axis_index("c")
        sid = lax.axis_index("s")
        wid = sid * NC + core
        dstv = (dst0, dst1)
        srcv = (src0, src1)
        rv = (r0, r1)
        fv = (f0, f1)
        slin = (sl0, sl1)
        sgat = (sg0, sg1)

        pltpu.sync_copy(z_hbm.at[pl.ds(sid * nr, nr)],
                        acc.at[pl.ds(sid * nr, nr)])
        plsc.subcore_barrier()

        def issue_linear(t, b):
            base = wid * ew + t * ch
            pltpu.async_copy(dst_hbm.at[pl.ds(base, ch)], dstv[b], slin[b])
            pltpu.async_copy(src_hbm.at[pl.ds(base, ch)], srcv[b], slin[b])
            pltpu.async_copy(fw2_hbm.at[pl.ds(base, ch)], fv[b], slin[b])

        def wait_linear(b):
            z = pl.ds(0, ch)
            pltpu.make_async_copy(dst_hbm.at[z], dstv[b], slin[b]).wait()
            pltpu.make_async_copy(src_hbm.at[z], srcv[b], slin[b]).wait()
            pltpu.make_async_copy(fw2_hbm.at[z], fv[b], slin[b]).wait()

        def issue_gather(b):
            pltpu.async_copy(so2_hbm.at[dstv[b]], rv[b], sgat[b])

        def wait_gather(b):
            pltpu.make_async_copy(so2_hbm.at[dstv[b]], rv[b], sgat[b]).wait()

        def compute_scatter(b):
            def body(i, _):
                for j in range(h // LANES):
                    sl = pl.ds(LANES * j, LANES)
                    rv[b][i, sl] = rv[b][i, sl] * fv[b][i, sl]
                return 0
            lax.fori_loop(0, ch, body, 0)
            pltpu.sync_copy(rv[b], acc.at[srcv[b]], add=True)

        issue_linear(0, 0)
        wait_linear(0)
        issue_gather(0)
        issue_linear(1, 1)

        def step(q, _):
            t = 2 * q
            guard = q < nt // 2 - 1
            wait_gather(0)
            wait_linear(1)
            issue_gather(1)
            compute_scatter(0)

            @pl.when(guard)
            def _():
                issue_linear(t + 2, 0)
            wait_gather(1)

            @pl.when(guard)
            def _():
                wait_linear(0)
                issue_gather(0)
            compute_scatter(1)

            @pl.when(guard)
            def _():
                issue_linear(t + 3, 1)
            return 0
        lax.fori_loop(0, nt // 2, step, 0)

        plsc.subcore_barrier()
        pltpu.sync_copy(acc.at[pl.ds(sid * nr, nr)],
                        out_hbm.at[pl.ds(core * na + sid * nr, nr)])

    return k(so2, fw2, src, dst, zeros)


def _sc_vec_pass(p, so3, fw1, fw3, diff_c, dist, src, dst, zeros):
    n, h = p.shape
    na = zeros.shape[0]
    e = src.shape[0]
    ew = e // NW
    ch = 32
    nt = ew // ch
    nr = na // NS
    mesh = plsc.VectorSubcoreMesh(core_axis_name="c", subcore_axis_name="s",
                                  num_cores=NC, num_subcores=NS)

    @functools.partial(
        pl.kernel, mesh=mesh,
        out_type=jax.ShapeDtypeStruct((NC * na, h), jnp.float32),
        scratch_types=[
            pltpu.VMEM((ch,), jnp.int32), pltpu.VMEM((ch,), jnp.int32),
            pltpu.VMEM((ch,), jnp.int32), pltpu.VMEM((ch,), jnp.int32),
            pltpu.VMEM((ch, h), jnp.float32), pltpu.VMEM((ch, h), jnp.float32),
            pltpu.VMEM((ch, h), jnp.float32), pltpu.VMEM((ch, h), jnp.float32),
            pltpu.VMEM((ch, h), jnp.float32), pltpu.VMEM((ch, h), jnp.float32),
            pltpu.VMEM((ch, h), jnp.float32), pltpu.VMEM((ch, h), jnp.float32),
            pltpu.VMEM((ch,), jnp.float32), pltpu.VMEM((ch,), jnp.float32),
            pltpu.VMEM((ch,), jnp.float32), pltpu.VMEM((ch,), jnp.float32),
            pltpu.VMEM_SHARED((na, h), jnp.float32),
            pltpu.SemaphoreType.DMA, pltpu.SemaphoreType.DMA,
            pltpu.SemaphoreType.DMA, pltpu.SemaphoreType.DMA,
        ],
    )
    def k(p_hbm, so3_hbm, fw1_hbm, fw3_hbm, diff_hbm, dist_hbm, src_hbm,
          dst_hbm, z_hbm, out_hbm,
          dst0, dst1, src0, src1, p0, p1, so30, so31, fw10, fw11,
          fw30, fw31, df0, df1, ds0, ds1, acc, sl0, sl1, sg0, sg1):
        core = lax.axis_index("c")
        sid = lax.axis_index("s")
        wid = sid * NC + core
        dstv = (dst0, dst1)
        srcv = (src0, src1)
        pv = (p0, p1)
        so3v = (so30, so31)
        fw1v = (fw10, fw11)
        fw3v = (fw30, fw31)
        dfv = (df0, df1)
        dsv = (ds0, ds1)
        slin = (sl0, sl1)
        sgat = (sg0, sg1)

        pltpu.sync_copy(z_hbm.at[pl.ds(sid * nr, nr)],
                        acc.at[pl.ds(sid * nr, nr)])
        plsc.subcore_barrier()

        def issue_linear(t, b):
            base = wid * ew + t * ch
            pltpu.async_copy(dst_hbm.at[pl.ds(base, ch)], dstv[b], slin[b])
            pltpu.async_copy(src_hbm.at[pl.ds(base, ch)], srcv[b], slin[b])
            pltpu.async_copy(fw1_hbm.at[pl.ds(base, ch)], fw1v[b], slin[b])
            pltpu.async_copy(fw3_hbm.at[pl.ds(base, ch)], fw3v[b], slin[b])
            pltpu.async_copy(diff_hbm.at[pl.ds(base, ch)], dfv[b], slin[b])
            pltpu.async_copy(dist_hbm.at[pl.ds(base, ch)], dsv[b], slin[b])

        def wait_linear(b):
            z = pl.ds(0, ch)
            pltpu.make_async_copy(dst_hbm.at[z], dstv[b], slin[b]).wait()
            pltpu.make_async_copy(src_hbm.at[z], srcv[b], slin[b]).wait()
            pltpu.make_async_copy(fw1_hbm.at[z], fw1v[b], slin[b]).wait()
            pltpu.make_async_copy(fw3_hbm.at[z], fw3v[b], slin[b]).wait()
            pltpu.make_async_copy(diff_hbm.at[z], dfv[b], slin[b]).wait()
            pltpu.make_async_copy(dist_hbm.at[z], dsv[b], slin[b]).wait()

        def issue_gather(b):
            pltpu.async_copy(p_hbm.at[dstv[b]], pv[b], sgat[b])
            pltpu.async_copy(so3_hbm.at[dstv[b]], so3v[b], sgat[b])

        def wait_gather(b):
            pltpu.make_async_copy(p_hbm.at[dstv[b]], pv[b], sgat[b]).wait()
            pltpu.make_async_copy(so3_hbm.at[dstv[b]], so3v[b], sgat[b]).wait()

        def compute_scatter(b):
            def grp(g, _):
                gsl = pl.ds(LANES * g, LANES)
                udv = dfv[b][gsl] / dsv[b][gsl]
                for jj in range(LANES):
                    i = LANES * g + jj
                    bc = _lane_bcast(udv, jj)
                    for j in range(h // LANES):
                        sl = pl.ds(LANES * j, LANES)
                        pv[b][i, sl] = (pv[b][i, sl] * fw1v[b][i, sl]
                                        + bc * (so3v[b][i, sl] * fw3v[b][i, sl]))
                return 0
            lax.fori_loop(0, ch // LANES, grp, 0)
            pltpu.sync_copy(pv[b], acc.at[srcv[b]], add=True)

        issue_linear(0, 0)
        wait_linear(0)
        issue_gather(0)
        issue_linear(1, 1)

        def step(q, _):
            t = 2 * q
            guard = q < nt // 2 - 1
            wait_gather(0)
            wait_linear(1)
            issue_gather(1)
            compute_scatter(0)

            @pl.when(guard)
            def _():
                issue_linear(t + 2, 0)
            wait_gather(1)

            @pl.when(guard)
            def _():
                wait_linear(0)
                issue_gather(0)
            compute_scatter(1)

            @pl.when(guard)
            def _():
                issue_linear(t + 3, 1)
            return 0
        lax.fori_loop(0, nt // 2, step, 0)

        plsc.subcore_barrier()
        pltpu.sync_copy(acc.at[pl.ds(sid * nr, nr)],
                        out_hbm.at[pl.ds(core * na + sid * nr, nr)])

    return k(p, so3, fw1, fw3, diff_c, dist, src, dst, zeros)


def _final_body(ns, nv0, nv1, nv2, rs, rv0, rv1, rv2,
                os_, ov0, ov1, ov2):
    os_[...] = ns[...] + rs[0] + rs[1]
    ov0[...] = nv0[...] + rv0[0] + rv0[1]
    ov1[...] = nv1[...] + rv1[0] + rv1[1]
    ov2[...] = nv2[...] + rv2[0] + rv2[1]


def _final_add(ns, nv0, nv1, nv2, rs, rv0, rv1, rv2):
    n, h = ns.shape
    bn = 400
    grid = (n // bn,)
    row = lambda i: (i, 0)
    row3 = lambda i: (0, i, 0)
    out = jax.ShapeDtypeStruct((n, h), jnp.float32)
    return pl.pallas_call(
        _final_body,
        grid=grid,
        in_specs=[pl.BlockSpec((bn, h), row)] * 4
                 + [pl.BlockSpec((NC, bn, h), row3)] * 4,
        out_specs=[pl.BlockSpec((bn, h), row)] * 4,
        out_shape=[out] * 4,
    )(ns, nv0, nv1, nv2, rs, rv0, rv1, rv2)


def kernel(node_scalar, node_vector, edge, edge_diff, edge_dist, rbf_dist,
           W1, b1, a1, W2, b2, a2, Wf1, bf1, af1, Wf2, bf2, af2):
    n, h = node_scalar.shape
    e = edge.shape[0]
    e2 = -(-e // 2048) * 2048
    pad = e2 - e
    na = -(-(n + 1) // 128) * 128

    src = jnp.concatenate([edge[:, 0].astype(jnp.int32),
                           jnp.full((pad,), n, jnp.int32)])
    dst = jnp.concatenate([edge[:, 1].astype(jnp.int32),
                           jnp.zeros((pad,), jnp.int32)])
    rbf_p = jnp.concatenate([rbf_dist,
                             jnp.zeros((pad, rbf_dist.shape[1]), jnp.float32)])
    dist_p = jnp.concatenate([edge_dist, jnp.ones((pad,), jnp.float32)])
    diff_p = jnp.concatenate([edge_diff, jnp.zeros((pad, 3), jnp.float32)])
    zeros_acc = jnp.zeros((na, h), jnp.float32)
    nv0 = node_vector[:, 0, :]
    nv1 = node_vector[:, 1, :]
    nv2 = node_vector[:, 2, :]
    d0 = diff_p[:, 0]
    d1 = diff_p[:, 1]
    d2 = diff_p[:, 2]
    b1r = b1.reshape(1, h)
    b2r = b2.reshape(1, 3 * h)
    bf1r = bf1.reshape(1, h)
    bf2r = bf2.reshape(1, 3 * h)
    a1r = a1.reshape(1, 1)
    a2r = a2.reshape(1, 1)
    af1r = af1.reshape(1, 1)
    af2r = af2.reshape(1, 1)

    p0, p1, p2, so2, so3 = _node_mlp(node_scalar, nv0, nv1, nv2,
                                     W1, b1r, a1r, W2, b2r, a2r)
    fw1, fw2, fw3 = _filter_mlp(rbf_p, Wf1, bf1r, af1r, Wf2, bf2r, af2r)

    rs = _sc_scalar_pass(so2, fw2, src, dst, zeros_acc).reshape(NC, na, h)
    rv0 = _sc_vec_pass(p0, so3, fw1, fw3, d0, dist_p, src, dst,
                       zeros_acc).reshape(NC, na, h)
    rv1 = _sc_vec_pass(p1, so3, fw1, fw3, d1, dist_p, src, dst,
                       zeros_acc).reshape(NC, na, h)
    rv2 = _sc_vec_pass(p2, so3, fw1, fw3, d2, dist_p, src, dst,
                       zeros_acc).reshape(NC, na, h)

    os_, ov0, ov1, ov2 = _final_add(node_scalar, nv0, nv1, nv2,
                                    rs, rv0, rv1, rv2)
    out_vector = jnp.stack([ov0, ov1, ov2], axis=1)
    return (os_, out_vector)
